# baseline (speedup 1.0000x reference)
import functools

import jax
import jax.numpy as jnp
from jax import lax
from jax.experimental import pallas as pl
from jax.experimental.pallas import tpu as pltpu
from jax.experimental.pallas import tpu_sc as plsc

_D = 1024
_NC = 2
_NS = 16
_NW = _NC * _NS
_C = 8
_NBUF = 8


@functools.cache
def _build(n_rows):
    bpw = n_rows // _NW
    nchunk = bpw // _C
    assert nchunk % _NBUF == 0
    mesh = plsc.VectorSubcoreMesh(core_axis_name="c", subcore_axis_name="s")

    @functools.partial(
        pl.kernel,
        mesh=mesh,
        out_type=jax.ShapeDtypeStruct((n_rows, _D), jnp.float32),
        scratch_types=[
            pltpu.VMEM((nchunk, _C), jnp.int32),
        ]
        + [pltpu.VMEM((_C, _D), jnp.float32) for _ in range(_NBUF)]
        + [pltpu.SemaphoreType.DMA for _ in range(2 * _NBUF)],
    )
    def k(idx_hbm, table_hbm, out_hbm, idx_v, *bufs_sems):
        bufs = bufs_sems[:_NBUF]
        gsems = bufs_sems[_NBUF:2 * _NBUF]
        wsems = bufs_sems[2 * _NBUF:]
        wid = lax.axis_index("s") * _NC + lax.axis_index("c")
        pltpu.sync_copy(idx_hbm.at[wid], idx_v)
        base = wid * bpw

        def gather(j, b):
            return pltpu.async_copy(table_hbm.at[idx_v.at[j]], bufs[b], gsems[b])

        def write(j, b):
            return pltpu.async_copy(
                bufs[b], out_hbm.at[pl.ds(base + j * _C, _C)], wsems[b])

        for b in range(_NBUF):
            gather(b, b)

        def body(p, carry):
            for b in range(_NBUF):
                j = p * _NBUF + b
                pltpu.make_async_copy(
                    table_hbm.at[idx_v.at[j]], bufs[b], gsems[b]).wait()
                write(j, b)

                @pl.when(j + _NBUF < nchunk)
                def _():
                    pltpu.make_async_copy(
                        bufs[b], out_hbm.at[pl.ds(base + j * _C, _C)],
                        wsems[b]).wait()
                    gather(j + _NBUF, b)

            return carry

        lax.fori_loop(0, nchunk // _NBUF, body, 0)

        for b in range(_NBUF):
            j = nchunk - _NBUF + b
            pltpu.make_async_copy(
                bufs[b], out_hbm.at[pl.ds(base + j * _C, _C)], wsems[b]).wait()

    return k


def kernel(input, pe):
    b, s = input.shape
    n = b * s
    idx = input.reshape(_NW, (n // _NW) // _C, _C)
    out = _build(n)(idx, pe)
    return out.reshape(b, s, _D)
